# Initial kernel scaffold; baseline (speedup 1.0000x reference)
import jax
import jax.numpy as jnp
from jax import lax
from jax.experimental import pallas as pl
from jax.experimental.pallas import tpu as pltpu
from jax.experimental.pallas import tpu_sc as plsc

BATCH = 4096
HIST = 200
WORD_DIM = 32
B = BATCH * HIST
NW = 32
B_PER_W = B // NW
CHUNK = 3200
NCHUNK = B_PER_W // CHUNK


def _emb_body(idx_hbm, table_hbm, out_hbm, idx_v, rows_v, sem):
    wid = lax.axis_index("s") * 2 + lax.axis_index("c")
    base = wid * B_PER_W

    def step(c, _):
        off = pl.multiple_of(base + c * CHUNK, CHUNK)
        pltpu.sync_copy(idx_hbm.at[pl.ds(off, CHUNK)], idx_v)
        pltpu.async_copy(table_hbm.at[idx_v], rows_v, sem).wait()
        pltpu.sync_copy(rows_v, out_hbm.at[pl.ds(off, CHUNK)])
        return 0

    lax.fori_loop(0, NCHUNK, step, 0)


@jax.jit
def _emb(idx, word_em):
    return pl.kernel(
        _emb_body,
        out_type=jax.ShapeDtypeStruct((B, WORD_DIM), jnp.float32),
        mesh=plsc.VectorSubcoreMesh(core_axis_name="c", subcore_axis_name="s"),
        scratch_types=[
            pltpu.VMEM((CHUNK,), jnp.int32),
            pltpu.VMEM((CHUNK, WORD_DIM), jnp.float32),
            pltpu.SemaphoreType.DMA,
        ],
    )(idx, word_em)


def kernel(review, word_em):
    idx = review.reshape(B).astype(jnp.int32)
    out = _emb(idx, word_em)
    return out.reshape(BATCH, HIST, WORD_DIM)


# speedup vs baseline: 1.4945x; 1.4945x over previous
import jax
import jax.numpy as jnp
from jax import lax
from jax.experimental import pallas as pl
from jax.experimental.pallas import tpu as pltpu
from jax.experimental.pallas import tpu_sc as plsc

BATCH = 4096
HIST = 200
WORD_DIM = 32
B = BATCH * HIST
NW = 32
B_PER_W = B // NW
CHUNK = 3200
NCHUNK = B_PER_W // CHUNK


def _emb_body(idx_hbm, table_hbm, out_hbm, idx_v, rows_v, sem):
    wid = lax.axis_index("s") * 2 + lax.axis_index("c")
    base = wid * B_PER_W

    def step(c, _):
        off = pl.multiple_of(base + c * CHUNK, CHUNK)
        pltpu.sync_copy(idx_hbm.at[pl.ds(off, CHUNK)], idx_v)
        pltpu.async_copy(table_hbm.at[idx_v], rows_v, sem).wait()
        pltpu.sync_copy(rows_v, out_hbm.at[pl.ds(off, CHUNK)])
        return 0

    lax.fori_loop(0, NCHUNK, step, 0)


@jax.jit
def _emb(idx, word_em):
    return pl.kernel(
        _emb_body,
        out_type=jax.ShapeDtypeStruct((B, WORD_DIM), jnp.float32),
        mesh=plsc.VectorSubcoreMesh(core_axis_name="c", subcore_axis_name="s"),
        scratch_types=[
            pltpu.VMEM((CHUNK,), jnp.int32),
            pltpu.VMEM((CHUNK, WORD_DIM), jnp.float32),
            pltpu.SemaphoreType.DMA,
        ],
        compiler_params=pltpu.CompilerParams(use_tc_tiling_on_sc=False),
    )(idx, word_em)


def kernel(review, word_em):
    idx = review.reshape(B).astype(jnp.int32)
    out = _emb(idx, word_em)
    return out.reshape(BATCH, HIST, WORD_DIM)
